# baseline (speedup 1.0000x reference)
import functools

import jax
import jax.numpy as jnp
from jax import lax
from jax.experimental import pallas as pl
from jax.experimental.pallas import tpu as pltpu
from jax.experimental.pallas import tpu_sc as plsc

N_NODES = 10000
N_EDGES = 320000
B = 8
EQU = 32
INV = 16
HID = 64
NL = 4
ROWS_E = B * EQU
ROWS = ROWS_E + B * INV

WB = 1280
NBLK = N_EDGES // WB

GROUP = 6
K = 3200
NCHUNK = N_EDGES // K

_mesh = plsc.VectorSubcoreMesh(core_axis_name="c", subcore_axis_name="s")
_sc_params = pltpu.CompilerParams(needs_layout_passes=False)


def _sc_body(xt_hbm, src_hbm, dst_hbm, msg_hbm, tab_v, sbuf_v, dbuf_v, vbuf_v):
    wid = lax.axis_index("s") * 2 + lax.axis_index("c")
    for grp in range(12 // GROUP):
        base_row = wid * 12 + grp * GROUP

        def _zero(i, carry):
            tab_v[pl.ds(i * 16, 16)] = jnp.zeros((16,), jnp.float32)
            return carry

        lax.fori_loop(0, GROUP * N_NODES // 16, _zero, 0)

        def _chunk_scatter(c, carry):
            pltpu.sync_copy(src_hbm.at[pl.ds(c * K, K)], sbuf_v)
            pltpu.sync_copy(dst_hbm.at[pl.ds(c * K, K)], dbuf_v)
            for r in range(GROUP):
                pltpu.sync_copy(xt_hbm.at[base_row + r, pl.ds(c * K, K)],
                                vbuf_v.at[r])

            def _grp(g, carry2):
                s16 = sbuf_v[pl.ds(g * 16, 16)]
                d16 = dbuf_v[pl.ds(g * 16, 16)]
                for r in range(GROUP):
                    sgn = jnp.where(base_row + r < ROWS_E, -1.0, 1.0)
                    v = vbuf_v[r, pl.ds(g * 16, 16)]
                    plsc.addupdate_scatter(tab_v, [s16 + r * N_NODES], v)
                    plsc.addupdate_scatter(tab_v, [d16 + r * N_NODES], v * sgn)
                return carry2

            lax.fori_loop(0, K // 16, _grp, 0)
            return carry

        lax.fori_loop(0, NCHUNK, _chunk_scatter, 0)

        def _chunk_gather(c, carry):
            pltpu.sync_copy(src_hbm.at[pl.ds(c * K, K)], sbuf_v)
            pltpu.sync_copy(dst_hbm.at[pl.ds(c * K, K)], dbuf_v)

            def _grp(g, carry2):
                s16 = sbuf_v[pl.ds(g * 16, 16)]
                d16 = dbuf_v[pl.ds(g * 16, 16)]
                for r in range(GROUP):
                    sgn = jnp.where(base_row + r < ROWS_E, -1.0, 1.0)
                    ga = plsc.load_gather(tab_v, [s16 + r * N_NODES])
                    gb = plsc.load_gather(tab_v, [d16 + r * N_NODES])
                    vbuf_v[r, pl.ds(g * 16, 16)] = ga + sgn * gb
                return carry2

            lax.fori_loop(0, K // 16, _grp, 0)
            for r in range(GROUP):
                pltpu.sync_copy(vbuf_v.at[r],
                                msg_hbm.at[base_row + r, pl.ds(c * K, K)])
            return carry

        lax.fori_loop(0, NCHUNK, _chunk_gather, 0)


_sc_layer = functools.partial(
    pl.kernel,
    mesh=_mesh,
    compiler_params=_sc_params,
    out_type=jax.ShapeDtypeStruct((ROWS, N_EDGES), jnp.float32),
    scratch_types=[
        pltpu.VMEM((GROUP * N_NODES,), jnp.float32),
        pltpu.VMEM((K,), jnp.int32),
        pltpu.VMEM((K,), jnp.int32),
        pltpu.VMEM((GROUP, K), jnp.float32),
    ],
)(_sc_body)


def _prep_body(li_ref, aux_ref, af_ref, wequ_ref, bequ_ref, wi0_ref, wi1_ref,
               binv_ref, xt_ref, mf_ref):
    m = li_ref[...].astype(jnp.float32)
    mf_ref[...] = m
    a = af_ref[:, 0:1]
    wequ = wequ_ref[:, 0:1]
    bequ = bequ_ref[:, 0:1]
    sigma = aux_ref[0:1, :]
    for b in range(B):
        xt_ref[b * EQU:(b + 1) * EQU, :] = sigma * wequ + bequ
        flip = jnp.where(a[b:b + 1, :] > 0.5,
                         1.0 - 2.0 * m[b:b + 1, :],
                         jnp.ones_like(sigma))
        sigma = sigma * flip
    xi = (aux_ref[1:2, :] * wi0_ref[:, 0:1] + aux_ref[2:3, :] * wi1_ref[:, 0:1]
          + binv_ref[:, 0:1])
    for b in range(B):
        xt_ref[ROWS_E + b * INV:ROWS_E + (b + 1) * INV, :] = xi


def _upd_body(xt_ref, mg_ref, weT_ref, wiT_ref, wcT_ref, bi_ref, out_ref):
    bi = bi_ref[:, 0:1]
    weT = weT_ref[...]
    wiT = wiT_ref[...]
    wcT = wcT_ref[...]
    for b in range(B):
        xe = xt_ref[b * EQU:(b + 1) * EQU, :]
        me = mg_ref[b * EQU:(b + 1) * EQU, :]
        xe_new = xe + jnp.tanh(
            jax.lax.dot(weT, me, preferred_element_type=jnp.float32))
        out_ref[b * EQU:(b + 1) * EQU, :] = xe_new
        lo = ROWS_E + b * INV
        xi = xt_ref[lo:lo + INV, :]
        mi = mg_ref[lo:lo + INV, :]
        pre = (xi + jax.lax.dot(wiT, mi, preferred_element_type=jnp.float32)
               + bi
               + jax.lax.dot(wcT, xe_new * xe_new,
                             preferred_element_type=jnp.float32))
        out_ref[lo:lo + INV, :] = jax.nn.gelu(pre)


def _pool_body(xt_ref, mf_ref, pp_ref, cnt_ref):
    i = pl.program_id(0)

    @pl.when(i == 0)
    def _():
        pp_ref[...] = jnp.zeros_like(pp_ref)
        cnt_ref[...] = jnp.zeros_like(cnt_ref)

    m = mf_ref[...]
    for j in range(WB // 128):
        cnt_ref[...] += m[:, j * 128:(j + 1) * 128]
    for b in range(B):
        mb = m[b:b + 1, :]
        xe = xt_ref[b * EQU:(b + 1) * EQU, :] * mb
        lo = ROWS_E + b * INV
        xi = xt_ref[lo:lo + INV, :] * mb
        for j in range(WB // 128):
            pp_ref[b * EQU:(b + 1) * EQU, :] += xe[:, j * 128:(j + 1) * 128]
            pp_ref[lo:lo + INV, :] += xi[:, j * 128:(j + 1) * 128]


def _head_body(pe_ref, pi_ref, cnt_ref, af_ref, h0e_ref, h0i_ref, b0_ref,
               h1_ref, b1_ref, h2_ref, b2_ref, out_ref):
    pe = jnp.sum(pe_ref[...], axis=2)
    pi = jnp.sum(pi_ref[...], axis=2)
    n = jnp.maximum(jnp.sum(cnt_ref[...], axis=1, keepdims=True), 1.0)
    pe = pe / n
    pi = pi / n
    h = jax.nn.gelu(
        jax.lax.dot(pe, h0e_ref[...], preferred_element_type=jnp.float32)
        + jax.lax.dot(pi, h0i_ref[...], preferred_element_type=jnp.float32)
        + b0_ref[...])
    h = jax.nn.gelu(
        jax.lax.dot(h, h1_ref[...], preferred_element_type=jnp.float32)
        + b1_ref[...])
    logit = (jax.lax.dot(h, h2_ref[...], preferred_element_type=jnp.float32)
             + b2_ref[0:1, 0:1])
    p = jnp.clip(jax.nn.sigmoid(logit), 1e-06, 1.0 - 1e-06)
    a = af_ref[:, 0:1]
    lp = a * jnp.log(p) + (1.0 - a) * jnp.log(1.0 - p)
    out_ref[...] = jnp.full((8, 128), jnp.sum(lp), jnp.float32)


def _full(shape):
    return pl.BlockSpec(shape, lambda i: (0, 0))


_prep = pl.pallas_call(
    _prep_body,
    grid=(NBLK,),
    in_specs=[
        pl.BlockSpec((B, WB), lambda i: (0, i)),
        pl.BlockSpec((B, WB), lambda i: (0, i)),
        _full((B, 128)), _full((EQU, 128)), _full((EQU, 128)),
        _full((INV, 128)), _full((INV, 128)), _full((INV, 128)),
    ],
    out_specs=[pl.BlockSpec((ROWS, WB), lambda i: (0, i)),
               pl.BlockSpec((B, WB), lambda i: (0, i))],
    out_shape=[jax.ShapeDtypeStruct((ROWS, N_EDGES), jnp.float32),
               jax.ShapeDtypeStruct((B, N_EDGES), jnp.float32)],
)

_upd = pl.pallas_call(
    _upd_body,
    grid=(NBLK,),
    in_specs=[
        pl.BlockSpec((ROWS, WB), lambda i: (0, i)),
        pl.BlockSpec((ROWS, WB), lambda i: (0, i)),
        _full((EQU, EQU)), _full((INV, INV)), _full((INV, EQU)),
        _full((INV, 128)),
    ],
    out_specs=pl.BlockSpec((ROWS, WB), lambda i: (0, i)),
    out_shape=jax.ShapeDtypeStruct((ROWS, N_EDGES), jnp.float32),
)

_pool = pl.pallas_call(
    _pool_body,
    grid=(NBLK,),
    in_specs=[
        pl.BlockSpec((ROWS, WB), lambda i: (0, i)),
        pl.BlockSpec((B, WB), lambda i: (0, i)),
    ],
    out_specs=[_full((ROWS, 128)), _full((B, 128))],
    out_shape=[jax.ShapeDtypeStruct((ROWS, 128), jnp.float32),
               jax.ShapeDtypeStruct((B, 128), jnp.float32)],
)

_head = pl.pallas_call(
    _head_body,
    out_shape=jax.ShapeDtypeStruct((8, 128), jnp.float32),
)


def kernel(alpha, sigma_seed, inv_features, edge_index, loop_indicators,
           W_equ_in, b_equ_in, W_inv_in, b_inv_in, We, Wi, bi, Wc,
           H0, b0, H1, b1, H2, b2):
    src = edge_index[0]
    dst = edge_index[1]
    af = jnp.broadcast_to(alpha.astype(jnp.float32)[:, None], (B, 128))
    aux = jnp.concatenate(
        [sigma_seed[None, :], inv_features.T,
         jnp.zeros((B - 3, N_EDGES), jnp.float32)], axis=0)
    wequ = jnp.broadcast_to(W_equ_in[0][:, None], (EQU, 128))
    bequ = jnp.broadcast_to(b_equ_in[:, None], (EQU, 128))
    wi0 = jnp.broadcast_to(W_inv_in[0][:, None], (INV, 128))
    wi1 = jnp.broadcast_to(W_inv_in[1][:, None], (INV, 128))
    binv = jnp.broadcast_to(b_inv_in[:, None], (INV, 128))

    xt, mf = _prep(loop_indicators, aux, af, wequ, bequ, wi0, wi1, binv)
    for l in range(NL):
        msg = _sc_layer(xt, src, dst)
        xt = _upd(xt, msg, We[l].T, Wi[l].T, Wc[l].T,
                  jnp.broadcast_to(bi[l][:, None], (INV, 128)))
    pp, cnt = _pool(xt, mf)
    pe3 = pp[:ROWS_E].reshape(B, EQU, 128)
    pi3 = pp[ROWS_E:].reshape(B, INV, 128)
    out = _head(pe3, pi3, cnt, af, H0[:EQU], H0[EQU:], b0[None, :],
                H1, b1[None, :], H2,
                jnp.broadcast_to(b2[:, None], (1, 128)))
    return out[0, 0]


# speedup vs baseline: 1.3446x; 1.3446x over previous
import functools

import jax
import jax.numpy as jnp
from jax import lax
from jax.experimental import pallas as pl
from jax.experimental.pallas import tpu as pltpu
from jax.experimental.pallas import tpu_sc as plsc

N_NODES = 10000
N_EDGES = 320000
B = 8
EQU = 32
INV = 16
HID = 64
NL = 4
ROWS_E = B * EQU
ROWS = ROWS_E + B * INV

WB = 1280
NBLK = N_EDGES // WB

GROUP = 6
K = 3200
NCHUNK = N_EDGES // K

_mesh = plsc.VectorSubcoreMesh(core_axis_name="c", subcore_axis_name="s")
_sc_params = pltpu.CompilerParams(needs_layout_passes=False)


def _sc_body(xt_hbm, src_hbm, dst_hbm, msg_hbm,
             t0, t1, t2, t3, t4, t5, sbuf_v, dbuf_v, vbuf_v,
             sem0, sem1, semo0, semo1):
    wid = lax.axis_index("s") * 2 + lax.axis_index("c")
    tabs = [t0, t1, t2, t3, t4, t5]
    isem = [sem0, sem1]
    osem = [semo0, semo1]
    groups = [
        [(wid * 8 + r, True) for r in range(6)],
        [(wid * 8 + 6 + r, True) for r in range(2)]
        + [(ROWS_E + wid * 4 + j, False) for j in range(4)],
    ]

    for rows in groups:
        def _zero(i, carry):
            for t in tabs:
                t[pl.ds(i * 16, 16)] = jnp.zeros((16,), jnp.float32)
            return carry

        lax.fori_loop(0, N_NODES // 16, _zero, 0)

        def idx_copies(c, p):
            return [pltpu.make_async_copy(src_hbm.at[pl.ds(c * K, K)],
                                          sbuf_v.at[p], isem[p]),
                    pltpu.make_async_copy(dst_hbm.at[pl.ds(c * K, K)],
                                          dbuf_v.at[p], isem[p])]

        def val_copies(c, p):
            return [pltpu.make_async_copy(xt_hbm.at[row, pl.ds(c * K, K)],
                                          vbuf_v.at[p, r], isem[p])
                    for r, (row, _) in enumerate(rows)]

        def out_copies(c, p):
            return [pltpu.make_async_copy(vbuf_v.at[p, r],
                                          msg_hbm.at[row, pl.ds(c * K, K)],
                                          osem[p])
                    for r, (row, _) in enumerate(rows)]

        def start_in(c, p):
            for d in idx_copies(c, p) + val_copies(c, p):
                d.start()

        def wait_in(p):
            for d in idx_copies(0, p) + val_copies(0, p):
                d.wait()

        def compute_scatter(p):
            def _grp(g, carry):
                s16 = sbuf_v[p, pl.ds(g * 16, 16)]
                d16 = dbuf_v[p, pl.ds(g * 16, 16)]
                for r, (row, is_equ) in enumerate(rows):
                    v = vbuf_v[p, r, pl.ds(g * 16, 16)]
                    plsc.addupdate_scatter(tabs[r], [s16], v)
                    plsc.addupdate_scatter(tabs[r], [d16], -v if is_equ else v)
                return carry

            lax.fori_loop(0, K // 16, _grp, 0)

        start_in(0, 0)

        def sbody(i, carry):
            c0 = 2 * i
            wait_in(0)
            start_in(c0 + 1, 1)
            compute_scatter(0)
            wait_in(1)
            start_in(lax.rem(c0 + 2, NCHUNK), 0)
            compute_scatter(1)
            return carry

        lax.fori_loop(0, NCHUNK // 2, sbody, 0)
        wait_in(0)

        def start_idx(c, p):
            for d in idx_copies(c, p):
                d.start()

        def wait_idx(p):
            for d in idx_copies(0, p):
                d.wait()

        def compute_gather(p):
            def _grp(g, carry):
                s16 = sbuf_v[p, pl.ds(g * 16, 16)]
                d16 = dbuf_v[p, pl.ds(g * 16, 16)]
                for r, (row, is_equ) in enumerate(rows):
                    ga = plsc.load_gather(tabs[r], [s16])
                    gb = plsc.load_gather(tabs[r], [d16])
                    vbuf_v[p, r, pl.ds(g * 16, 16)] = (ga - gb if is_equ
                                                       else ga + gb)
                return carry

            lax.fori_loop(0, K // 16, _grp, 0)

        start_idx(0, 0)

        def gbody(i, carry):
            for half in (0, 1):
                c = 2 * i + half
                wait_idx(half)
                start_idx(lax.rem(c + 1, NCHUNK), 1 - half)

                @pl.when(i >= 1)
                def _():
                    for d in out_copies(0, half):
                        d.wait()

                compute_gather(half)
                for d in out_copies(c, half):
                    d.start()
            return carry

        lax.fori_loop(0, NCHUNK // 2, gbody, 0)
        wait_idx(0)
        for p in (0, 1):
            for d in out_copies(0, p):
                d.wait()


_sc_layer = functools.partial(
    pl.kernel,
    mesh=_mesh,
    compiler_params=_sc_params,
    out_type=jax.ShapeDtypeStruct((ROWS, N_EDGES), jnp.float32),
    scratch_types=(
        [pltpu.VMEM((N_NODES,), jnp.float32)] * 6
        + [pltpu.VMEM((2, K), jnp.int32),
           pltpu.VMEM((2, K), jnp.int32),
           pltpu.VMEM((2, 6, K), jnp.float32),
           pltpu.SemaphoreType.DMA,
           pltpu.SemaphoreType.DMA,
           pltpu.SemaphoreType.DMA,
           pltpu.SemaphoreType.DMA]
    ),
)(_sc_body)


def _prep_body(li_ref, aux_ref, af_ref, wequ_ref, bequ_ref, wi0_ref, wi1_ref,
               binv_ref, xt_ref, mf_ref):
    m = li_ref[...].astype(jnp.float32)
    mf_ref[...] = m
    a = af_ref[:, 0:1]
    wequ = wequ_ref[:, 0:1]
    bequ = bequ_ref[:, 0:1]
    sigma = aux_ref[0:1, :]
    for b in range(B):
        xt_ref[b * EQU:(b + 1) * EQU, :] = sigma * wequ + bequ
        flip = jnp.where(a[b:b + 1, :] > 0.5,
                         1.0 - 2.0 * m[b:b + 1, :],
                         jnp.ones_like(sigma))
        sigma = sigma * flip
    xi = (aux_ref[1:2, :] * wi0_ref[:, 0:1] + aux_ref[2:3, :] * wi1_ref[:, 0:1]
          + binv_ref[:, 0:1])
    for b in range(B):
        xt_ref[ROWS_E + b * INV:ROWS_E + (b + 1) * INV, :] = xi


def _upd_body(xt_ref, mg_ref, weT_ref, wiT_ref, wcT_ref, bi_ref, out_ref):
    bi = bi_ref[:, 0:1]
    weT = weT_ref[...]
    wiT = wiT_ref[...]
    wcT = wcT_ref[...]
    for b in range(B):
        xe = xt_ref[b * EQU:(b + 1) * EQU, :]
        me = mg_ref[b * EQU:(b + 1) * EQU, :]
        xe_new = xe + jnp.tanh(
            jax.lax.dot(weT, me, preferred_element_type=jnp.float32))
        out_ref[b * EQU:(b + 1) * EQU, :] = xe_new
        lo = ROWS_E + b * INV
        xi = xt_ref[lo:lo + INV, :]
        mi = mg_ref[lo:lo + INV, :]
        pre = (xi + jax.lax.dot(wiT, mi, preferred_element_type=jnp.float32)
               + bi
               + jax.lax.dot(wcT, xe_new * xe_new,
                             preferred_element_type=jnp.float32))
        out_ref[lo:lo + INV, :] = jax.nn.gelu(pre)


def _pool_body(xt_ref, mf_ref, pp_ref, cnt_ref):
    i = pl.program_id(0)

    @pl.when(i == 0)
    def _():
        pp_ref[...] = jnp.zeros_like(pp_ref)
        cnt_ref[...] = jnp.zeros_like(cnt_ref)

    m = mf_ref[...]
    for j in range(WB // 128):
        cnt_ref[...] += m[:, j * 128:(j + 1) * 128]
    for b in range(B):
        mb = m[b:b + 1, :]
        xe = xt_ref[b * EQU:(b + 1) * EQU, :] * mb
        lo = ROWS_E + b * INV
        xi = xt_ref[lo:lo + INV, :] * mb
        for j in range(WB // 128):
            pp_ref[b * EQU:(b + 1) * EQU, :] += xe[:, j * 128:(j + 1) * 128]
            pp_ref[lo:lo + INV, :] += xi[:, j * 128:(j + 1) * 128]


def _head_body(pe_ref, pi_ref, cnt_ref, af_ref, h0e_ref, h0i_ref, b0_ref,
               h1_ref, b1_ref, h2_ref, b2_ref, out_ref):
    pe = jnp.sum(pe_ref[...], axis=2)
    pi = jnp.sum(pi_ref[...], axis=2)
    n = jnp.maximum(jnp.sum(cnt_ref[...], axis=1, keepdims=True), 1.0)
    pe = pe / n
    pi = pi / n
    h = jax.nn.gelu(
        jax.lax.dot(pe, h0e_ref[...], preferred_element_type=jnp.float32)
        + jax.lax.dot(pi, h0i_ref[...], preferred_element_type=jnp.float32)
        + b0_ref[...])
    h = jax.nn.gelu(
        jax.lax.dot(h, h1_ref[...], preferred_element_type=jnp.float32)
        + b1_ref[...])
    logit = (jax.lax.dot(h, h2_ref[...], preferred_element_type=jnp.float32)
             + b2_ref[0:1, 0:1])
    p = jnp.clip(jax.nn.sigmoid(logit), 1e-06, 1.0 - 1e-06)
    a = af_ref[:, 0:1]
    lp = a * jnp.log(p) + (1.0 - a) * jnp.log(1.0 - p)
    out_ref[...] = jnp.full((8, 128), jnp.sum(lp), jnp.float32)


def _full(shape):
    return pl.BlockSpec(shape, lambda i: (0, 0))


_prep = pl.pallas_call(
    _prep_body,
    grid=(NBLK,),
    in_specs=[
        pl.BlockSpec((B, WB), lambda i: (0, i)),
        pl.BlockSpec((B, WB), lambda i: (0, i)),
        _full((B, 128)), _full((EQU, 128)), _full((EQU, 128)),
        _full((INV, 128)), _full((INV, 128)), _full((INV, 128)),
    ],
    out_specs=[pl.BlockSpec((ROWS, WB), lambda i: (0, i)),
               pl.BlockSpec((B, WB), lambda i: (0, i))],
    out_shape=[jax.ShapeDtypeStruct((ROWS, N_EDGES), jnp.float32),
               jax.ShapeDtypeStruct((B, N_EDGES), jnp.float32)],
)

_upd = pl.pallas_call(
    _upd_body,
    grid=(NBLK,),
    in_specs=[
        pl.BlockSpec((ROWS, WB), lambda i: (0, i)),
        pl.BlockSpec((ROWS, WB), lambda i: (0, i)),
        _full((EQU, EQU)), _full((INV, INV)), _full((INV, EQU)),
        _full((INV, 128)),
    ],
    out_specs=pl.BlockSpec((ROWS, WB), lambda i: (0, i)),
    out_shape=jax.ShapeDtypeStruct((ROWS, N_EDGES), jnp.float32),
)

_pool = pl.pallas_call(
    _pool_body,
    grid=(NBLK,),
    in_specs=[
        pl.BlockSpec((ROWS, WB), lambda i: (0, i)),
        pl.BlockSpec((B, WB), lambda i: (0, i)),
    ],
    out_specs=[_full((ROWS, 128)), _full((B, 128))],
    out_shape=[jax.ShapeDtypeStruct((ROWS, 128), jnp.float32),
               jax.ShapeDtypeStruct((B, 128), jnp.float32)],
)

_head = pl.pallas_call(
    _head_body,
    out_shape=jax.ShapeDtypeStruct((8, 128), jnp.float32),
)


def kernel(alpha, sigma_seed, inv_features, edge_index, loop_indicators,
           W_equ_in, b_equ_in, W_inv_in, b_inv_in, We, Wi, bi, Wc,
           H0, b0, H1, b1, H2, b2):
    src = edge_index[0]
    dst = edge_index[1]
    af = jnp.broadcast_to(alpha.astype(jnp.float32)[:, None], (B, 128))
    aux = jnp.concatenate(
        [sigma_seed[None, :], inv_features.T,
         jnp.zeros((B - 3, N_EDGES), jnp.float32)], axis=0)
    wequ = jnp.broadcast_to(W_equ_in[0][:, None], (EQU, 128))
    bequ = jnp.broadcast_to(b_equ_in[:, None], (EQU, 128))
    wi0 = jnp.broadcast_to(W_inv_in[0][:, None], (INV, 128))
    wi1 = jnp.broadcast_to(W_inv_in[1][:, None], (INV, 128))
    binv = jnp.broadcast_to(b_inv_in[:, None], (INV, 128))

    xt, mf = _prep(loop_indicators, aux, af, wequ, bequ, wi0, wi1, binv)
    for l in range(NL):
        msg = _sc_layer(xt, src, dst)
        xt = _upd(xt, msg, We[l].T, Wi[l].T, Wc[l].T,
                  jnp.broadcast_to(bi[l][:, None], (INV, 128)))
    pp, cnt = _pool(xt, mf)
    pe3 = pp[:ROWS_E].reshape(B, EQU, 128)
    pi3 = pp[ROWS_E:].reshape(B, INV, 128)
    out = _head(pe3, pi3, cnt, af, H0[:EQU], H0[EQU:], b0[None, :],
                H1, b1[None, :], H2,
                jnp.broadcast_to(b2[:, None], (1, 128)))
    return out[0, 0]


# speedup vs baseline: 2.3926x; 1.7794x over previous
import functools

import jax
import jax.numpy as jnp
from jax import lax
from jax.experimental import pallas as pl
from jax.experimental.pallas import tpu as pltpu
from jax.experimental.pallas import tpu_sc as plsc

N_NODES = 10000
N_EDGES = 320000
B = 8
EQU = 32
INV = 16
HID = 64
NL = 4
ROWS_E = B * EQU
ROWS = ROWS_E + B * INV

WB = 1280
NBLK = N_EDGES // WB

GROUP = 6
K = 3200
NCHUNK = N_EDGES // K

_mesh = plsc.VectorSubcoreMesh(core_axis_name="c", subcore_axis_name="s")
_sc_params = pltpu.CompilerParams(needs_layout_passes=False)


def _sc_body(xt_hbm, src_hbm, dst_hbm, msg_hbm,
             t0, t1, t2, t3, t4, t5, sbuf_v, dbuf_v, vbuf_v,
             sem0, sem1, semo0, semo1):
    wid = lax.axis_index("s") * 2 + lax.axis_index("c")
    tabs = [t0, t1, t2, t3, t4, t5]
    isem = [sem0, sem1]
    osem = [semo0, semo1]
    groups = [
        [(wid * 8 + r, True) for r in range(6)],
        [(wid * 8 + 6 + r, True) for r in range(2)]
        + [(ROWS_E + wid * 4 + j, False) for j in range(4)],
    ]

    for rows in groups:
        @plsc.parallel_loop(0, N_NODES // 16, unroll=4)
        def _zero(i):
            for t in tabs:
                t[pl.ds(i * 16, 16)] = jnp.zeros((16,), jnp.float32)

        def idx_copies(c, p):
            return [pltpu.make_async_copy(src_hbm.at[pl.ds(c * K, K)],
                                          sbuf_v.at[p], isem[p]),
                    pltpu.make_async_copy(dst_hbm.at[pl.ds(c * K, K)],
                                          dbuf_v.at[p], isem[p])]

        def val_copies(c, p):
            return [pltpu.make_async_copy(xt_hbm.at[row, pl.ds(c * K, K)],
                                          vbuf_v.at[p, r], isem[p])
                    for r, (row, _) in enumerate(rows)]

        def out_copies(c, p):
            return [pltpu.make_async_copy(vbuf_v.at[p, r],
                                          msg_hbm.at[row, pl.ds(c * K, K)],
                                          osem[p])
                    for r, (row, _) in enumerate(rows)]

        def start_in(c, p):
            for d in idx_copies(c, p) + val_copies(c, p):
                d.start()

        def wait_in(p):
            for d in idx_copies(0, p) + val_copies(0, p):
                d.wait()

        def compute_scatter(p):
            @plsc.parallel_loop(0, K // 16, unroll=2)
            def _grp(g):
                s16 = sbuf_v[p, pl.ds(g * 16, 16)]
                d16 = dbuf_v[p, pl.ds(g * 16, 16)]
                for r, (row, is_equ) in enumerate(rows):
                    v = vbuf_v[p, r, pl.ds(g * 16, 16)]
                    plsc.addupdate_scatter(tabs[r], [s16], v)
                    plsc.addupdate_scatter(tabs[r], [d16], -v if is_equ else v)

        start_in(0, 0)

        def sbody(i, carry):
            c0 = 2 * i
            wait_in(0)
            start_in(c0 + 1, 1)
            compute_scatter(0)
            wait_in(1)
            start_in(lax.rem(c0 + 2, NCHUNK), 0)
            compute_scatter(1)
            return carry

        lax.fori_loop(0, NCHUNK // 2, sbody, 0)
        wait_in(0)

        def start_idx(c, p):
            for d in idx_copies(c, p):
                d.start()

        def wait_idx(p):
            for d in idx_copies(0, p):
                d.wait()

        def compute_gather(p):
            @plsc.parallel_loop(0, K // 16, unroll=2)
            def _grp(g):
                s16 = sbuf_v[p, pl.ds(g * 16, 16)]
                d16 = dbuf_v[p, pl.ds(g * 16, 16)]
                for r, (row, is_equ) in enumerate(rows):
                    ga = plsc.load_gather(tabs[r], [s16])
                    gb = plsc.load_gather(tabs[r], [d16])
                    vbuf_v[p, r, pl.ds(g * 16, 16)] = (ga - gb if is_equ
                                                       else ga + gb)

        start_idx(0, 0)

        def gbody(i, carry):
            for half in (0, 1):
                c = 2 * i + half
                wait_idx(half)
                start_idx(lax.rem(c + 1, NCHUNK), 1 - half)

                @pl.when(i >= 1)
                def _():
                    for d in out_copies(0, half):
                        d.wait()

                compute_gather(half)
                for d in out_copies(c, half):
                    d.start()
            return carry

        lax.fori_loop(0, NCHUNK // 2, gbody, 0)
        wait_idx(0)
        for p in (0, 1):
            for d in out_copies(0, p):
                d.wait()


_sc_layer = functools.partial(
    pl.kernel,
    mesh=_mesh,
    compiler_params=_sc_params,
    out_type=jax.ShapeDtypeStruct((ROWS, N_EDGES), jnp.float32),
    scratch_types=(
        [pltpu.VMEM((N_NODES,), jnp.float32)] * 6
        + [pltpu.VMEM((2, K), jnp.int32),
           pltpu.VMEM((2, K), jnp.int32),
           pltpu.VMEM((2, 6, K), jnp.float32),
           pltpu.SemaphoreType.DMA,
           pltpu.SemaphoreType.DMA,
           pltpu.SemaphoreType.DMA,
           pltpu.SemaphoreType.DMA]
    ),
)(_sc_body)


def _prep_body(li_ref, aux_ref, af_ref, wequ_ref, bequ_ref, wi0_ref, wi1_ref,
               binv_ref, xt_ref, mf_ref):
    m = li_ref[...].astype(jnp.float32)
    mf_ref[...] = m
    a = af_ref[:, 0:1]
    wequ = wequ_ref[:, 0:1]
    bequ = bequ_ref[:, 0:1]
    sigma = aux_ref[0:1, :]
    for b in range(B):
        xt_ref[b * EQU:(b + 1) * EQU, :] = sigma * wequ + bequ
        flip = jnp.where(a[b:b + 1, :] > 0.5,
                         1.0 - 2.0 * m[b:b + 1, :],
                         jnp.ones_like(sigma))
        sigma = sigma * flip
    xi = (aux_ref[1:2, :] * wi0_ref[:, 0:1] + aux_ref[2:3, :] * wi1_ref[:, 0:1]
          + binv_ref[:, 0:1])
    for b in range(B):
        xt_ref[ROWS_E + b * INV:ROWS_E + (b + 1) * INV, :] = xi


def _upd_body(xt_ref, mg_ref, weT_ref, wiT_ref, wcT_ref, bi_ref, out_ref):
    bi = bi_ref[:, 0:1]
    weT = weT_ref[...]
    wiT = wiT_ref[...]
    wcT = wcT_ref[...]
    for b in range(B):
        xe = xt_ref[b * EQU:(b + 1) * EQU, :]
        me = mg_ref[b * EQU:(b + 1) * EQU, :]
        xe_new = xe + jnp.tanh(
            jax.lax.dot(weT, me, preferred_element_type=jnp.float32))
        out_ref[b * EQU:(b + 1) * EQU, :] = xe_new
        lo = ROWS_E + b * INV
        xi = xt_ref[lo:lo + INV, :]
        mi = mg_ref[lo:lo + INV, :]
        pre = (xi + jax.lax.dot(wiT, mi, preferred_element_type=jnp.float32)
               + bi
               + jax.lax.dot(wcT, xe_new * xe_new,
                             preferred_element_type=jnp.float32))
        out_ref[lo:lo + INV, :] = jax.nn.gelu(pre)


def _pool_body(xt_ref, mf_ref, pp_ref, cnt_ref):
    i = pl.program_id(0)

    @pl.when(i == 0)
    def _():
        pp_ref[...] = jnp.zeros_like(pp_ref)
        cnt_ref[...] = jnp.zeros_like(cnt_ref)

    m = mf_ref[...]
    for j in range(WB // 128):
        cnt_ref[...] += m[:, j * 128:(j + 1) * 128]
    for b in range(B):
        mb = m[b:b + 1, :]
        xe = xt_ref[b * EQU:(b + 1) * EQU, :] * mb
        lo = ROWS_E + b * INV
        xi = xt_ref[lo:lo + INV, :] * mb
        for j in range(WB // 128):
            pp_ref[b * EQU:(b + 1) * EQU, :] += xe[:, j * 128:(j + 1) * 128]
            pp_ref[lo:lo + INV, :] += xi[:, j * 128:(j + 1) * 128]


def _head_body(pe_ref, pi_ref, cnt_ref, af_ref, h0e_ref, h0i_ref, b0_ref,
               h1_ref, b1_ref, h2_ref, b2_ref, out_ref):
    pe = jnp.sum(pe_ref[...], axis=2)
    pi = jnp.sum(pi_ref[...], axis=2)
    n = jnp.maximum(jnp.sum(cnt_ref[...], axis=1, keepdims=True), 1.0)
    pe = pe / n
    pi = pi / n
    h = jax.nn.gelu(
        jax.lax.dot(pe, h0e_ref[...], preferred_element_type=jnp.float32)
        + jax.lax.dot(pi, h0i_ref[...], preferred_element_type=jnp.float32)
        + b0_ref[...])
    h = jax.nn.gelu(
        jax.lax.dot(h, h1_ref[...], preferred_element_type=jnp.float32)
        + b1_ref[...])
    logit = (jax.lax.dot(h, h2_ref[...], preferred_element_type=jnp.float32)
             + b2_ref[0:1, 0:1])
    p = jnp.clip(jax.nn.sigmoid(logit), 1e-06, 1.0 - 1e-06)
    a = af_ref[:, 0:1]
    lp = a * jnp.log(p) + (1.0 - a) * jnp.log(1.0 - p)
    out_ref[...] = jnp.full((8, 128), jnp.sum(lp), jnp.float32)


def _full(shape):
    return pl.BlockSpec(shape, lambda i: (0, 0))


_prep = pl.pallas_call(
    _prep_body,
    grid=(NBLK,),
    in_specs=[
        pl.BlockSpec((B, WB), lambda i: (0, i)),
        pl.BlockSpec((B, WB), lambda i: (0, i)),
        _full((B, 128)), _full((EQU, 128)), _full((EQU, 128)),
        _full((INV, 128)), _full((INV, 128)), _full((INV, 128)),
    ],
    out_specs=[pl.BlockSpec((ROWS, WB), lambda i: (0, i)),
               pl.BlockSpec((B, WB), lambda i: (0, i))],
    out_shape=[jax.ShapeDtypeStruct((ROWS, N_EDGES), jnp.float32),
               jax.ShapeDtypeStruct((B, N_EDGES), jnp.float32)],
)

_upd = pl.pallas_call(
    _upd_body,
    grid=(NBLK,),
    in_specs=[
        pl.BlockSpec((ROWS, WB), lambda i: (0, i)),
        pl.BlockSpec((ROWS, WB), lambda i: (0, i)),
        _full((EQU, EQU)), _full((INV, INV)), _full((INV, EQU)),
        _full((INV, 128)),
    ],
    out_specs=pl.BlockSpec((ROWS, WB), lambda i: (0, i)),
    out_shape=jax.ShapeDtypeStruct((ROWS, N_EDGES), jnp.float32),
)

_pool = pl.pallas_call(
    _pool_body,
    grid=(NBLK,),
    in_specs=[
        pl.BlockSpec((ROWS, WB), lambda i: (0, i)),
        pl.BlockSpec((B, WB), lambda i: (0, i)),
    ],
    out_specs=[_full((ROWS, 128)), _full((B, 128))],
    out_shape=[jax.ShapeDtypeStruct((ROWS, 128), jnp.float32),
               jax.ShapeDtypeStruct((B, 128), jnp.float32)],
)

_head = pl.pallas_call(
    _head_body,
    out_shape=jax.ShapeDtypeStruct((8, 128), jnp.float32),
)


def kernel(alpha, sigma_seed, inv_features, edge_index, loop_indicators,
           W_equ_in, b_equ_in, W_inv_in, b_inv_in, We, Wi, bi, Wc,
           H0, b0, H1, b1, H2, b2):
    src = edge_index[0]
    dst = edge_index[1]
    af = jnp.broadcast_to(alpha.astype(jnp.float32)[:, None], (B, 128))
    aux = jnp.concatenate(
        [sigma_seed[None, :], inv_features.T,
         jnp.zeros((B - 3, N_EDGES), jnp.float32)], axis=0)
    wequ = jnp.broadcast_to(W_equ_in[0][:, None], (EQU, 128))
    bequ = jnp.broadcast_to(b_equ_in[:, None], (EQU, 128))
    wi0 = jnp.broadcast_to(W_inv_in[0][:, None], (INV, 128))
    wi1 = jnp.broadcast_to(W_inv_in[1][:, None], (INV, 128))
    binv = jnp.broadcast_to(b_inv_in[:, None], (INV, 128))

    xt, mf = _prep(loop_indicators, aux, af, wequ, bequ, wi0, wi1, binv)
    for l in range(NL):
        msg = _sc_layer(xt, src, dst)
        xt = _upd(xt, msg, We[l].T, Wi[l].T, Wc[l].T,
                  jnp.broadcast_to(bi[l][:, None], (INV, 128)))
    pp, cnt = _pool(xt, mf)
    pe3 = pp[:ROWS_E].reshape(B, EQU, 128)
    pi3 = pp[ROWS_E:].reshape(B, INV, 128)
    out = _head(pe3, pi3, cnt, af, H0[:EQU], H0[EQU:], b0[None, :],
                H1, b1[None, :], H2,
                jnp.broadcast_to(b2[:, None], (1, 128)))
    return out[0, 0]


# speedup vs baseline: 2.3990x; 1.0026x over previous
import functools

import jax
import jax.numpy as jnp
from jax import lax
from jax.experimental import pallas as pl
from jax.experimental.pallas import tpu as pltpu
from jax.experimental.pallas import tpu_sc as plsc

N_NODES = 10000
N_EDGES = 320000
B = 8
EQU = 32
INV = 16
HID = 64
NL = 4
ROWS_E = B * EQU
ROWS = ROWS_E + B * INV

WB = 1280
NBLK = N_EDGES // WB

GROUP = 6
K = 3200
NCHUNK = N_EDGES // K

_mesh = plsc.VectorSubcoreMesh(core_axis_name="c", subcore_axis_name="s")
_sc_params = pltpu.CompilerParams(needs_layout_passes=False)


def _sc_body(xt_hbm, src_hbm, dst_hbm, msg_hbm,
             t0, t1, t2, t3, t4, t5, sbuf_v, dbuf_v, vbuf_v,
             sem0, sem1, semo0, semo1):
    wid = lax.axis_index("s") * 2 + lax.axis_index("c")
    tabs = [t0, t1, t2, t3, t4, t5]
    isem = [sem0, sem1]
    osem = [semo0, semo1]
    groups = [
        [(wid * 8 + r, True) for r in range(6)],
        [(wid * 8 + 6 + r, True) for r in range(2)]
        + [(ROWS_E + wid * 4 + j, False) for j in range(4)],
    ]

    for rows in groups:
        @plsc.parallel_loop(0, N_NODES // 16, unroll=4)
        def _zero(i):
            for t in tabs:
                t[pl.ds(i * 16, 16)] = jnp.zeros((16,), jnp.float32)

        def idx_copies(c, p):
            return [pltpu.make_async_copy(src_hbm.at[pl.ds(c * K, K)],
                                          sbuf_v.at[p], isem[p]),
                    pltpu.make_async_copy(dst_hbm.at[pl.ds(c * K, K)],
                                          dbuf_v.at[p], isem[p])]

        def val_copies(c, p):
            return [pltpu.make_async_copy(xt_hbm.at[row, pl.ds(c * K, K)],
                                          vbuf_v.at[p, r], isem[p])
                    for r, (row, _) in enumerate(rows)]

        def out_copies(c, p):
            return [pltpu.make_async_copy(vbuf_v.at[p, r],
                                          msg_hbm.at[row, pl.ds(c * K, K)],
                                          osem[p])
                    for r, (row, _) in enumerate(rows)]

        def start_in(c, p):
            for d in idx_copies(c, p) + val_copies(c, p):
                d.start()

        def wait_in(p):
            for d in idx_copies(0, p) + val_copies(0, p):
                d.wait()

        def compute_scatter(p):
            @plsc.parallel_loop(0, K // 16, unroll=4)
            def _grp(g):
                s16 = sbuf_v[p, pl.ds(g * 16, 16)]
                d16 = dbuf_v[p, pl.ds(g * 16, 16)]
                for r, (row, is_equ) in enumerate(rows):
                    v = vbuf_v[p, r, pl.ds(g * 16, 16)]
                    plsc.addupdate_scatter(tabs[r], [s16], v)
                    plsc.addupdate_scatter(tabs[r], [d16], -v if is_equ else v)

        start_in(0, 0)

        def sbody(i, carry):
            c0 = 2 * i
            wait_in(0)
            start_in(c0 + 1, 1)
            compute_scatter(0)
            wait_in(1)
            start_in(lax.rem(c0 + 2, NCHUNK), 0)
            compute_scatter(1)
            return carry

        lax.fori_loop(0, NCHUNK // 2, sbody, 0)
        wait_in(0)

        def start_idx(c, p):
            for d in idx_copies(c, p):
                d.start()

        def wait_idx(p):
            for d in idx_copies(0, p):
                d.wait()

        def compute_gather(p):
            @plsc.parallel_loop(0, K // 16, unroll=4)
            def _grp(g):
                s16 = sbuf_v[p, pl.ds(g * 16, 16)]
                d16 = dbuf_v[p, pl.ds(g * 16, 16)]
                for r, (row, is_equ) in enumerate(rows):
                    ga = plsc.load_gather(tabs[r], [s16])
                    gb = plsc.load_gather(tabs[r], [d16])
                    vbuf_v[p, r, pl.ds(g * 16, 16)] = (ga - gb if is_equ
                                                       else ga + gb)

        start_idx(0, 0)

        def gbody(i, carry):
            for half in (0, 1):
                c = 2 * i + half
                wait_idx(half)
                start_idx(lax.rem(c + 1, NCHUNK), 1 - half)

                @pl.when(i >= 1)
                def _():
                    for d in out_copies(0, half):
                        d.wait()

                compute_gather(half)
                for d in out_copies(c, half):
                    d.start()
            return carry

        lax.fori_loop(0, NCHUNK // 2, gbody, 0)
        wait_idx(0)
        for p in (0, 1):
            for d in out_copies(0, p):
                d.wait()


_sc_layer = functools.partial(
    pl.kernel,
    mesh=_mesh,
    compiler_params=_sc_params,
    out_type=jax.ShapeDtypeStruct((ROWS, N_EDGES), jnp.float32),
    scratch_types=(
        [pltpu.VMEM((N_NODES,), jnp.float32)] * 6
        + [pltpu.VMEM((2, K), jnp.int32),
           pltpu.VMEM((2, K), jnp.int32),
           pltpu.VMEM((2, 6, K), jnp.float32),
           pltpu.SemaphoreType.DMA,
           pltpu.SemaphoreType.DMA,
           pltpu.SemaphoreType.DMA,
           pltpu.SemaphoreType.DMA]
    ),
)(_sc_body)


def _prep_body(li_ref, aux_ref, af_ref, wequ_ref, bequ_ref, wi0_ref, wi1_ref,
               binv_ref, xt_ref, mf_ref):
    m = li_ref[...].astype(jnp.float32)
    mf_ref[...] = m
    a = af_ref[:, 0:1]
    wequ = wequ_ref[:, 0:1]
    bequ = bequ_ref[:, 0:1]
    sigma = aux_ref[0:1, :]
    for b in range(B):
        xt_ref[b * EQU:(b + 1) * EQU, :] = sigma * wequ + bequ
        flip = jnp.where(a[b:b + 1, :] > 0.5,
                         1.0 - 2.0 * m[b:b + 1, :],
                         jnp.ones_like(sigma))
        sigma = sigma * flip
    xi = (aux_ref[1:2, :] * wi0_ref[:, 0:1] + aux_ref[2:3, :] * wi1_ref[:, 0:1]
          + binv_ref[:, 0:1])
    for b in range(B):
        xt_ref[ROWS_E + b * INV:ROWS_E + (b + 1) * INV, :] = xi


def _upd_body(xt_ref, mg_ref, weT_ref, wiT_ref, wcT_ref, bi_ref, out_ref):
    bi = bi_ref[:, 0:1]
    weT = weT_ref[...]
    wiT = wiT_ref[...]
    wcT = wcT_ref[...]
    for b in range(B):
        xe = xt_ref[b * EQU:(b + 1) * EQU, :]
        me = mg_ref[b * EQU:(b + 1) * EQU, :]
        xe_new = xe + jnp.tanh(
            jax.lax.dot(weT, me, preferred_element_type=jnp.float32))
        out_ref[b * EQU:(b + 1) * EQU, :] = xe_new
        lo = ROWS_E + b * INV
        xi = xt_ref[lo:lo + INV, :]
        mi = mg_ref[lo:lo + INV, :]
        pre = (xi + jax.lax.dot(wiT, mi, preferred_element_type=jnp.float32)
               + bi
               + jax.lax.dot(wcT, xe_new * xe_new,
                             preferred_element_type=jnp.float32))
        out_ref[lo:lo + INV, :] = jax.nn.gelu(pre)


def _pool_body(xt_ref, mf_ref, pp_ref, cnt_ref):
    i = pl.program_id(0)

    @pl.when(i == 0)
    def _():
        pp_ref[...] = jnp.zeros_like(pp_ref)
        cnt_ref[...] = jnp.zeros_like(cnt_ref)

    m = mf_ref[...]
    for j in range(WB // 128):
        cnt_ref[...] += m[:, j * 128:(j + 1) * 128]
    for b in range(B):
        mb = m[b:b + 1, :]
        xe = xt_ref[b * EQU:(b + 1) * EQU, :] * mb
        lo = ROWS_E + b * INV
        xi = xt_ref[lo:lo + INV, :] * mb
        for j in range(WB // 128):
            pp_ref[b * EQU:(b + 1) * EQU, :] += xe[:, j * 128:(j + 1) * 128]
            pp_ref[lo:lo + INV, :] += xi[:, j * 128:(j + 1) * 128]


def _head_body(pe_ref, pi_ref, cnt_ref, af_ref, h0e_ref, h0i_ref, b0_ref,
               h1_ref, b1_ref, h2_ref, b2_ref, out_ref):
    pe = jnp.sum(pe_ref[...], axis=2)
    pi = jnp.sum(pi_ref[...], axis=2)
    n = jnp.maximum(jnp.sum(cnt_ref[...], axis=1, keepdims=True), 1.0)
    pe = pe / n
    pi = pi / n
    h = jax.nn.gelu(
        jax.lax.dot(pe, h0e_ref[...], preferred_element_type=jnp.float32)
        + jax.lax.dot(pi, h0i_ref[...], preferred_element_type=jnp.float32)
        + b0_ref[...])
    h = jax.nn.gelu(
        jax.lax.dot(h, h1_ref[...], preferred_element_type=jnp.float32)
        + b1_ref[...])
    logit = (jax.lax.dot(h, h2_ref[...], preferred_element_type=jnp.float32)
             + b2_ref[0:1, 0:1])
    p = jnp.clip(jax.nn.sigmoid(logit), 1e-06, 1.0 - 1e-06)
    a = af_ref[:, 0:1]
    lp = a * jnp.log(p) + (1.0 - a) * jnp.log(1.0 - p)
    out_ref[...] = jnp.full((8, 128), jnp.sum(lp), jnp.float32)


def _full(shape):
    return pl.BlockSpec(shape, lambda i: (0, 0))


_prep = pl.pallas_call(
    _prep_body,
    grid=(NBLK,),
    in_specs=[
        pl.BlockSpec((B, WB), lambda i: (0, i)),
        pl.BlockSpec((B, WB), lambda i: (0, i)),
        _full((B, 128)), _full((EQU, 128)), _full((EQU, 128)),
        _full((INV, 128)), _full((INV, 128)), _full((INV, 128)),
    ],
    out_specs=[pl.BlockSpec((ROWS, WB), lambda i: (0, i)),
               pl.BlockSpec((B, WB), lambda i: (0, i))],
    out_shape=[jax.ShapeDtypeStruct((ROWS, N_EDGES), jnp.float32),
               jax.ShapeDtypeStruct((B, N_EDGES), jnp.float32)],
)

_upd = pl.pallas_call(
    _upd_body,
    grid=(NBLK,),
    in_specs=[
        pl.BlockSpec((ROWS, WB), lambda i: (0, i)),
        pl.BlockSpec((ROWS, WB), lambda i: (0, i)),
        _full((EQU, EQU)), _full((INV, INV)), _full((INV, EQU)),
        _full((INV, 128)),
    ],
    out_specs=pl.BlockSpec((ROWS, WB), lambda i: (0, i)),
    out_shape=jax.ShapeDtypeStruct((ROWS, N_EDGES), jnp.float32),
)

_pool = pl.pallas_call(
    _pool_body,
    grid=(NBLK,),
    in_specs=[
        pl.BlockSpec((ROWS, WB), lambda i: (0, i)),
        pl.BlockSpec((B, WB), lambda i: (0, i)),
    ],
    out_specs=[_full((ROWS, 128)), _full((B, 128))],
    out_shape=[jax.ShapeDtypeStruct((ROWS, 128), jnp.float32),
               jax.ShapeDtypeStruct((B, 128), jnp.float32)],
)

_head = pl.pallas_call(
    _head_body,
    out_shape=jax.ShapeDtypeStruct((8, 128), jnp.float32),
)


def kernel(alpha, sigma_seed, inv_features, edge_index, loop_indicators,
           W_equ_in, b_equ_in, W_inv_in, b_inv_in, We, Wi, bi, Wc,
           H0, b0, H1, b1, H2, b2):
    src = edge_index[0]
    dst = edge_index[1]
    af = jnp.broadcast_to(alpha.astype(jnp.float32)[:, None], (B, 128))
    aux = jnp.concatenate(
        [sigma_seed[None, :], inv_features.T,
         jnp.zeros((B - 3, N_EDGES), jnp.float32)], axis=0)
    wequ = jnp.broadcast_to(W_equ_in[0][:, None], (EQU, 128))
    bequ = jnp.broadcast_to(b_equ_in[:, None], (EQU, 128))
    wi0 = jnp.broadcast_to(W_inv_in[0][:, None], (INV, 128))
    wi1 = jnp.broadcast_to(W_inv_in[1][:, None], (INV, 128))
    binv = jnp.broadcast_to(b_inv_in[:, None], (INV, 128))

    xt, mf = _prep(loop_indicators, aux, af, wequ, bequ, wi0, wi1, binv)
    for l in range(NL):
        msg = _sc_layer(xt, src, dst)
        xt = _upd(xt, msg, We[l].T, Wi[l].T, Wc[l].T,
                  jnp.broadcast_to(bi[l][:, None], (INV, 128)))
    pp, cnt = _pool(xt, mf)
    pe3 = pp[:ROWS_E].reshape(B, EQU, 128)
    pi3 = pp[ROWS_E:].reshape(B, INV, 128)
    out = _head(pe3, pi3, cnt, af, H0[:EQU], H0[EQU:], b0[None, :],
                H1, b1[None, :], H2,
                jnp.broadcast_to(b2[:, None], (1, 128)))
    return out[0, 0]
